# Initial kernel scaffold; baseline (speedup 1.0000x reference)
import jax
import jax.numpy as jnp
from jax.experimental import pallas as pl


def kernel(y_true, y_pred, B):
    raise NotImplementedError("write your pallas kernel here")


# speedup vs baseline: 4.6115x; 4.6115x over previous
import jax
import jax.numpy as jnp
from jax import lax
from jax.experimental import pallas as pl
from jax.experimental.pallas import tpu as pltpu

_TB = 2048


def _tc_body(yt_ref, yp_ref, bt_ref, out_ref):
    i = pl.program_id(0)

    @pl.when(i == 0)
    def _():
        out_ref[...] = jnp.zeros((1, 1), jnp.float32)

    idx = yt_ref[...]
    cls = lax.broadcasted_iota(jnp.int32, (_TB, 24), 1)
    oh = (idx == cls).astype(jnp.float32)
    gathered = jnp.dot(oh, bt_ref[...], preferred_element_type=jnp.float32)
    part = jnp.sum(gathered * yp_ref[...])
    out_ref[...] += jnp.reshape(part, (1, 1))


def kernel(y_true, y_pred, B):
    n = y_true.shape[0] * y_true.shape[1]
    yt = y_true.reshape(n, 1).astype(jnp.int32)
    yp = y_pred.reshape(n, y_pred.shape[-1])
    bt = jnp.transpose(B, (1, 0))
    grid = n // _TB
    out = pl.pallas_call(
        _tc_body,
        grid=(grid,),
        in_specs=[
            pl.BlockSpec((_TB, 1), lambda i: (i, 0)),
            pl.BlockSpec((_TB, 24), lambda i: (i, 0)),
            pl.BlockSpec((24, 24), lambda i: (0, 0)),
        ],
        out_specs=pl.BlockSpec((1, 1), lambda i: (0, 0)),
        out_shape=jax.ShapeDtypeStruct((1, 1), jnp.float32),
    )(yt, yp, bt)
    return out[0, 0]


# speedup vs baseline: 5.2664x; 1.1420x over previous
import functools

import jax
import jax.numpy as jnp
from jax import lax
from jax.experimental import pallas as pl
from jax.experimental.pallas import tpu as pltpu
from jax.experimental.pallas import tpu_sc as plsc

_N = 16384 * 200
_K = 24
_NC = 2
_NS = 16
_NW = _NC * _NS
_TPW = _N // _NW
_C = 1024
_NCHUNK = _TPW // _C
_G = _C // 16
_NACC = 4


def _compute_chunk(idx_ref, yp_ref, bt_ref, accs, iota24):
    def grp(g, accs):
        vi = idx_ref[pl.ds(g * 16, 16)]
        bbase = vi * _K
        tbase = g * (16 * _K) + iota24
        accs = list(accs)
        for c in range(_K):
            bv = plsc.load_gather(bt_ref, [bbase + c])
            ypv = plsc.load_gather(yp_ref, [tbase + c])
            accs[c % _NACC] = accs[c % _NACC] + ypv * bv
        return tuple(accs)

    return lax.fori_loop(0, _G, grp, accs)


def _sc_body(yp_hbm, yt_hbm, bt_hbm, out_hbm,
             bt_v, i0_v, i1_v, y0_v, y1_v, acc_v, sem0, sem1):
    cid = lax.axis_index("c")
    sid = lax.axis_index("s")
    wid = sid * _NC + cid
    tok0 = wid * _TPW

    pltpu.sync_copy(bt_hbm, bt_v)

    idx_bufs = (i0_v, i1_v)
    yp_bufs = (y0_v, y1_v)
    sems = (sem0, sem1)

    def start(chunk, b):
        t = tok0 + chunk * _C
        pltpu.async_copy(yt_hbm.at[pl.ds(t, _C)], idx_bufs[b], sems[b])
        pltpu.async_copy(yp_hbm.at[pl.ds(t * _K, _C * _K)], yp_bufs[b], sems[b])

    def wait(b):
        pltpu.make_async_copy(yt_hbm.at[pl.ds(0, _C)], idx_bufs[b], sems[b]).wait()
        pltpu.make_async_copy(yp_hbm.at[pl.ds(0, _C * _K)], yp_bufs[b], sems[b]).wait()

    for b in range(2):
        start(b, b)

    iota24 = lax.iota(jnp.int32, 16) * _K
    zero = jnp.zeros((16,), jnp.float32)
    accs = (zero, zero, zero, zero)

    def super_body(k, accs):
        for b in range(2):
            chunk = 2 * k + b
            wait(b)
            accs = _compute_chunk(idx_bufs[b], yp_bufs[b], bt_v, accs, iota24)

            @pl.when(chunk + 2 < _NCHUNK)
            def _():
                start(chunk + 2, b)
        return accs

    accs = lax.fori_loop(0, _NCHUNK // 2, super_body, accs)
    acc_v[...] = accs[0] + accs[1] + accs[2] + accs[3]
    pltpu.sync_copy(acc_v, out_hbm.at[wid])


@functools.partial(
    pl.kernel,
    mesh=plsc.VectorSubcoreMesh(core_axis_name="c", subcore_axis_name="s"),
    out_type=jax.ShapeDtypeStruct((_NW, 16), jnp.float32),
    compiler_params=pltpu.CompilerParams(needs_layout_passes=False),
    scratch_types=[
        pltpu.VMEM((_K * _K,), jnp.float32),
        pltpu.VMEM((_C,), jnp.int32),
        pltpu.VMEM((_C,), jnp.int32),
        pltpu.VMEM((_C * _K,), jnp.float32),
        pltpu.VMEM((_C * _K,), jnp.float32),
        pltpu.VMEM((16,), jnp.float32),
        pltpu.SemaphoreType.DMA,
        pltpu.SemaphoreType.DMA,
    ],
)
def _sc_kernel(yp_hbm, yt_hbm, bt_hbm, out_hbm, *scratch):
    _sc_body(yp_hbm, yt_hbm, bt_hbm, out_hbm, *scratch)


def kernel(y_true, y_pred, B):
    yt = y_true.reshape(_N).astype(jnp.int32)
    yp = y_pred.reshape(_N * _K)
    bt = jnp.transpose(B, (1, 0)).reshape(_K * _K)
    out = _sc_kernel(yp, yt, bt)
    return jnp.sum(out)


# speedup vs baseline: 48.0380x; 9.1216x over previous
import functools

import jax
import jax.numpy as jnp
from jax import lax
from jax.experimental import pallas as pl
from jax.experimental.pallas import tpu as pltpu
from jax.experimental.pallas import tpu_sc as plsc

_B = 16384
_T = 200
_K = 24
_NC = 2
_NS = 16
_NW = _NC * _NS
_BPW = _B // _NW
_G = _BPW // 16
_NACC = 4


def _compute_step(idx_ref, yp_ref, bt_ref, accs):
    def grp(g, accs):
        vi = idx_ref[pl.ds(g * 16, 16)]
        bbase = vi * _K
        accs = list(accs)
        for c in range(_K):
            bv = plsc.load_gather(bt_ref, [bbase + c])
            ypv = yp_ref[c, pl.ds(g * 16, 16)]
            accs[c % _NACC] = accs[c % _NACC] + ypv * bv
        return tuple(accs)

    return lax.fori_loop(0, _G, grp, accs)


def _sc_body(yp_hbm, yt_hbm, bt_hbm, out_hbm,
             bt_v, i0_v, i1_v, y0_v, y1_v, acc_v, sem0, sem1):
    cid = lax.axis_index("c")
    sid = lax.axis_index("s")
    wid = sid * _NC + cid
    i0 = wid * _BPW

    pltpu.sync_copy(bt_hbm, bt_v)

    idx_bufs = (i0_v, i1_v)
    yp_bufs = (y0_v, y1_v)
    sems = (sem0, sem1)

    def start(t, b):
        pltpu.async_copy(yt_hbm.at[t, pl.ds(i0, _BPW)], idx_bufs[b], sems[b])
        pltpu.async_copy(yp_hbm.at[t, :, pl.ds(i0, _BPW)], yp_bufs[b], sems[b])

    def wait(b):
        pltpu.make_async_copy(yt_hbm.at[0, pl.ds(0, _BPW)], idx_bufs[b], sems[b]).wait()
        pltpu.make_async_copy(yp_hbm.at[0, :, pl.ds(0, _BPW)], yp_bufs[b], sems[b]).wait()

    for b in range(2):
        start(b, b)

    zero = jnp.zeros((16,), jnp.float32)
    accs = (zero, zero, zero, zero)

    def super_body(k, accs):
        for b in range(2):
            t = 2 * k + b
            wait(b)
            accs = _compute_step(idx_bufs[b], yp_bufs[b], bt_v, accs)

            @pl.when(t + 2 < _T)
            def _():
                start(t + 2, b)
        return accs

    accs = lax.fori_loop(0, _T // 2, super_body, accs)
    acc_v[...] = accs[0] + accs[1] + accs[2] + accs[3]
    pltpu.sync_copy(acc_v, out_hbm.at[wid])


@functools.partial(
    pl.kernel,
    mesh=plsc.VectorSubcoreMesh(core_axis_name="c", subcore_axis_name="s"),
    out_type=jax.ShapeDtypeStruct((_NW, 16), jnp.float32),
    compiler_params=pltpu.CompilerParams(needs_layout_passes=False),
    scratch_types=[
        pltpu.VMEM((_K * _K,), jnp.float32),
        pltpu.VMEM((_BPW,), jnp.int32),
        pltpu.VMEM((_BPW,), jnp.int32),
        pltpu.VMEM((_K, _BPW), jnp.float32),
        pltpu.VMEM((_K, _BPW), jnp.float32),
        pltpu.VMEM((16,), jnp.float32),
        pltpu.SemaphoreType.DMA,
        pltpu.SemaphoreType.DMA,
    ],
)
def _sc_kernel(yp_hbm, yt_hbm, bt_hbm, out_hbm, *scratch):
    _sc_body(yp_hbm, yt_hbm, bt_hbm, out_hbm, *scratch)


def kernel(y_true, y_pred, B):
    ypt = jnp.transpose(y_pred, (1, 2, 0))
    ytt = jnp.transpose(y_true.astype(jnp.int32), (1, 0))
    bt = jnp.transpose(B, (1, 0)).reshape(_K * _K)
    out = _sc_kernel(ypt, ytt, bt)
    return jnp.sum(out)
